# baseline (speedup 1.0000x reference)
import functools

import jax
import jax.numpy as jnp
from jax import lax
from jax.experimental import pallas as pl
from jax.experimental.pallas import tpu as pltpu
from jax.experimental.pallas import tpu_sc as plsc

N = 4096
H = 768
E = 64
K = 2
MOE_FF = 768
FF = 1536

TM = 128
NT_PAD = (N * K) // TM + E
NP = NT_PAD * TM

TR = 512
TB = 128
TC_ = 512


def _router_body(x_ref, gw_ref, logits_ref, topv_ref, topi_ref, counts_ref,
                 acc_ref):
    i = pl.program_id(0)

    @pl.when(i == 0)
    def _():
        acc_ref[...] = jnp.zeros_like(acc_ref)

    x = x_ref[...]
    gw = gw_ref[...]
    logits = lax.dot_general(x, gw, (((1,), (1,)), ((), ())),
                             preferred_element_type=jnp.float32)
    logits_ref[...] = logits
    m = jnp.max(logits, axis=1, keepdims=True)
    p = jnp.exp(logits - m)
    probs = p / jnp.sum(p, axis=1, keepdims=True)
    cols = lax.broadcasted_iota(jnp.int32, probs.shape, 1)
    v1 = jnp.max(probs, axis=1, keepdims=True)
    i1 = jnp.min(jnp.where(probs == v1, cols, E), axis=1, keepdims=True)
    probs2 = jnp.where(cols == i1, -1.0, probs)
    v2 = jnp.max(probs2, axis=1, keepdims=True)
    i2 = jnp.min(jnp.where(probs2 == v2, cols, E), axis=1, keepdims=True)
    topv_ref[...] = jnp.concatenate([v1, v2], axis=1)
    topi_ref[...] = jnp.concatenate([i1, i2], axis=1)
    oh = ((cols == i1).astype(jnp.float32) + (cols == i2).astype(jnp.float32))
    acc_ref[...] = acc_ref[...] + jnp.sum(oh, axis=0, keepdims=True)
    counts_ref[...] = jnp.broadcast_to(acc_ref[...], counts_ref.shape)


def _router(x, gate_w):
    return pl.pallas_call(
        _router_body,
        grid=(N // TR,),
        in_specs=[
            pl.BlockSpec((TR, H), lambda i: (i, 0)),
            pl.BlockSpec((E, H), lambda i: (0, 0)),
        ],
        out_specs=[
            pl.BlockSpec((TR, E), lambda i: (i, 0)),
            pl.BlockSpec((TR, K), lambda i: (i, 0)),
            pl.BlockSpec((TR, K), lambda i: (i, 0)),
            pl.BlockSpec((8, E), lambda i: (0, 0)),
        ],
        out_shape=[
            jax.ShapeDtypeStruct((N, E), jnp.float32),
            jax.ShapeDtypeStruct((N, K), jnp.float32),
            jax.ShapeDtypeStruct((N, K), jnp.int32),
            jax.ShapeDtypeStruct((8, E), jnp.float32),
        ],
        scratch_shapes=[pltpu.VMEM((1, E), jnp.float32)],
        interpret=False,
    )(x, gate_w)


def _meta_body(topi_ref, counts_ref, ppos_ref, packed_ref, carry_ref,
               pstart_ref):
    s = pl.program_id(0)

    @pl.when(s == 0)
    def _():
        counts = counts_ref[0:1, :]
        counts_i = counts.astype(jnp.int32)
        tiles_i = (counts_i + (TM - 1)) // TM
        tiles_f = tiles_i.astype(jnp.float32)
        r64 = lax.broadcasted_iota(jnp.int32, (E, E), 0)
        c64 = lax.broadcasted_iota(jnp.int32, (E, E), 1)
        incl = (r64 <= c64).astype(jnp.float32)
        cum = lax.dot_general(tiles_f, incl, (((1,), (0,)), ((), ())),
                              preferred_element_type=jnp.float32)
        pstart_ref[...] = (cum - tiles_f) * TM
        carry_ref[...] = jnp.zeros_like(carry_ref)
        jrow = lax.broadcasted_iota(
            jnp.int32, (NT_PAD + 8, E), 0).astype(jnp.float32)
        te = jnp.sum((cum <= jrow).astype(jnp.float32), axis=1, keepdims=True)
        te = jnp.minimum(te, E - 1)
        total = jnp.sum(tiles_f)
        rid = lax.broadcasted_iota(jnp.int32, (NT_PAD + 8, 1), 0)
        packed_ref[...] = jnp.where(rid == NT_PAD, total, te).astype(jnp.int32)

    blk = topi_ref[...]
    e0 = blk[:, 0:1]
    e1 = blk[:, 1:2]
    lanes = lax.broadcasted_iota(jnp.int32, (TB, E), 1)
    oh0 = (lanes == e0).astype(jnp.float32)
    oh1 = (lanes == e1).astype(jnp.float32)
    r = lax.broadcasted_iota(jnp.int32, (TB, TB), 0)
    c = lax.broadcasted_iota(jnp.int32, (TB, TB), 1)
    ls = (c < r).astype(jnp.float32)
    base = carry_ref[...] + pstart_ref[...]
    rank0 = lax.dot_general(ls, oh0, (((1,), (0,)), ((), ())),
                            preferred_element_type=jnp.float32)
    pos0 = jnp.sum((rank0 + base) * oh0, axis=1, keepdims=True)
    sum0 = jnp.sum(oh0, axis=0, keepdims=True)
    rank1 = lax.dot_general(ls, oh1, (((1,), (0,)), ((), ())),
                            preferred_element_type=jnp.float32)
    pos1 = jnp.sum((rank1 + base + sum0) * oh1, axis=1, keepdims=True)
    ppos_ref[...] = jnp.concatenate([pos0, pos1], axis=0).astype(jnp.int32)
    carry_ref[...] = (carry_ref[...] + sum0
                      + jnp.sum(oh1, axis=0, keepdims=True))


def _meta(topi, counts8):
    return pl.pallas_call(
        _meta_body,
        grid=(N // TB,),
        in_specs=[
            pl.BlockSpec((TB, K), lambda s: (s, 0)),
            pl.BlockSpec((8, E), lambda s: (0, 0)),
        ],
        out_specs=[
            pl.BlockSpec((2 * TB, 1), lambda s: (s, 0)),
            pl.BlockSpec((NT_PAD + 8, 1), lambda s: (0, 0)),
        ],
        out_shape=[
            jax.ShapeDtypeStruct((N * K, 1), jnp.int32),
            jax.ShapeDtypeStruct((NT_PAD + 8, 1), jnp.int32),
        ],
        scratch_shapes=[pltpu.VMEM((1, E), jnp.float32),
                        pltpu.VMEM((1, E), jnp.float32)],
        interpret=False,
    )(topi, counts8)


CHUNK = 64
NCHUNK = (N * K) // CHUNK
NW = 32
CPW = NCHUNK // NW


def _dispatch_body(x_hbm, ppos_hbm, out_hbm, idx0, idx1, rows0, rows1,
                   sem0, sem1):
    wid = lax.axis_index("s") * 2 + lax.axis_index("c")
    idx = (idx0, idx1)
    rows = (rows0, rows1)
    sems = (sem0, sem1)
    pending = [None, None]
    for c in range(CPW):
        b = c % 2
        if pending[b] is not None:
            pending[b].wait()
        g = wid * CPW + c
        t0 = (g // 4) * TB + (g % 2) * CHUNK
        pltpu.sync_copy(ppos_hbm.at[g], idx[b])
        pltpu.sync_copy(x_hbm.at[pl.ds(t0, CHUNK)], rows[b])
        pending[b] = pltpu.async_copy(rows[b], out_hbm.at[idx[b]], sems[b])
    for p in pending:
        if p is not None:
            p.wait()


_dispatch = functools.partial(
    pl.kernel,
    mesh=plsc.VectorSubcoreMesh(core_axis_name="c", subcore_axis_name="s"),
    out_type=jax.ShapeDtypeStruct((NP, H), jnp.float32),
    scratch_types=[
        pltpu.VMEM((CHUNK,), jnp.int32),
        pltpu.VMEM((CHUNK,), jnp.int32),
        pltpu.VMEM((CHUNK, H), jnp.float32),
        pltpu.VMEM((CHUNK, H), jnp.float32),
        pltpu.SemaphoreType.DMA,
        pltpu.SemaphoreType.DMA,
    ],
)(_dispatch_body)


def _gmm_body(tile_eid_ref, ntiles_ref, x_ref, wg_ref, wu_ref, wd_ref, y_ref):
    i = pl.program_id(0)

    @pl.when(i < ntiles_ref[0])
    def _():
        x = x_ref[...].astype(jnp.bfloat16)
        g = lax.dot_general(x, wg_ref[0].astype(jnp.bfloat16),
                            (((1,), (1,)), ((), ())),
                            preferred_element_type=jnp.float32)
        u = lax.dot_general(x, wu_ref[0].astype(jnp.bfloat16),
                            (((1,), (1,)), ((), ())),
                            preferred_element_type=jnp.float32)
        h = (g * jax.nn.sigmoid(g) * u).astype(jnp.bfloat16)
        y = lax.dot_general(h, wd_ref[0].astype(jnp.bfloat16),
                            (((1,), (1,)), ((), ())),
                            preferred_element_type=jnp.float32)
        y_ref[...] = y.astype(jnp.bfloat16)


def _gmm(x_sorted, wg, wu, wd, tile_eid, ntiles):
    def _ci(i, nt):
        return jnp.minimum(i, nt[0] - 1)

    grid_spec = pltpu.PrefetchScalarGridSpec(
        num_scalar_prefetch=2,
        grid=(NT_PAD,),
        in_specs=[
            pl.BlockSpec((TM, H), lambda i, eid, nt: (_ci(i, nt), 0)),
            pl.BlockSpec((1, MOE_FF, H),
                         lambda i, eid, nt: (eid[_ci(i, nt)], 0, 0)),
            pl.BlockSpec((1, MOE_FF, H),
                         lambda i, eid, nt: (eid[_ci(i, nt)], 0, 0)),
            pl.BlockSpec((1, H, MOE_FF),
                         lambda i, eid, nt: (eid[_ci(i, nt)], 0, 0)),
        ],
        out_specs=pl.BlockSpec((TM, H), lambda i, eid, nt: (_ci(i, nt), 0)),
    )
    return pl.pallas_call(
        _gmm_body,
        grid_spec=grid_spec,
        out_shape=jax.ShapeDtypeStruct((NP, H), jnp.bfloat16),
        interpret=False,
    )(tile_eid, ntiles, x_sorted, wg, wu, wd)


def _shexp_body(x_ref, sg_ref, su_ref, sd_ref, seg_ref, out_ref):
    x = x_ref[...]
    xb = x.astype(jnp.bfloat16)
    g = lax.dot_general(xb, sg_ref[...].astype(jnp.bfloat16),
                        (((1,), (1,)), ((), ())),
                        preferred_element_type=jnp.float32)
    u = lax.dot_general(xb, su_ref[...].astype(jnp.bfloat16),
                        (((1,), (1,)), ((), ())),
                        preferred_element_type=jnp.float32)
    h = (g * jax.nn.sigmoid(g) * u).astype(jnp.bfloat16)
    sh = lax.dot_general(h, sd_ref[...].astype(jnp.bfloat16),
                         (((1,), (1,)), ((), ())),
                         preferred_element_type=jnp.float32)
    gate = jax.nn.sigmoid(
        lax.dot_general(x, seg_ref[...], (((1,), (1,)), ((), ())),
                        preferred_element_type=jnp.float32))
    out_ref[...] = gate * sh


def _shexp(x, sg, su, sd, seg):
    return pl.pallas_call(
        _shexp_body,
        grid=(N // TC_,),
        in_specs=[
            pl.BlockSpec((TC_, H), lambda i: (i, 0)),
            pl.BlockSpec((FF, H), lambda i: (0, 0)),
            pl.BlockSpec((FF, H), lambda i: (0, 0)),
            pl.BlockSpec((H, FF), lambda i: (0, 0)),
            pl.BlockSpec((1, H), lambda i: (0, 0)),
        ],
        out_specs=pl.BlockSpec((TC_, H), lambda i: (i, 0)),
        out_shape=jax.ShapeDtypeStruct((N, H), jnp.float32),
        interpret=False,
    )(x, sg, su, sd, seg)


def _combine_body(y0_ref, y1_ref, topv_ref, shg_ref, out_ref):
    topv = topv_ref[...]
    y0 = y0_ref[...].astype(jnp.float32)
    y1 = y1_ref[...].astype(jnp.float32)
    out_ref[...] = topv[:, 0:1] * y0 + topv[:, 1:2] * y1 + shg_ref[...]


def _combine(y0, y1, topv, shg):
    return pl.pallas_call(
        _combine_body,
        grid=(N // TC_,),
        in_specs=[
            pl.BlockSpec((TC_, H), lambda i: (i, 0)),
            pl.BlockSpec((TC_, H), lambda i: (i, 0)),
            pl.BlockSpec((TC_, K), lambda i: (i, 0)),
            pl.BlockSpec((TC_, H), lambda i: (i, 0)),
        ],
        out_specs=pl.BlockSpec((TC_, H), lambda i: (i, 0)),
        out_shape=jax.ShapeDtypeStruct((N, H), jnp.float32),
        interpret=False,
    )(y0, y1, topv, shg)


def kernel(hidden_states, gate_w, expert_gate_w, expert_up_w, expert_down_w,
           shared_gate_w, shared_up_w, shared_down_w, shared_expert_gate_w):
    Bv, Sv, Hd = hidden_states.shape
    x = hidden_states.reshape(-1, Hd)

    logits, topv, topi, counts8 = _router(x, gate_w)
    ppos_col, packed = _meta(topi, counts8)

    ppos_r = ppos_col.reshape(N // TB, K, TB)
    p0 = ppos_r[:, 0, :].reshape(N)
    p1 = ppos_r[:, 1, :].reshape(N)
    tile_eid = packed[:NT_PAD, 0]
    ntiles = packed[NT_PAD:NT_PAD + 1, 0]

    x_sorted = _dispatch(x, ppos_col.reshape(NCHUNK, CHUNK))
    y_sorted = _gmm(x_sorted, expert_gate_w, expert_up_w, expert_down_w,
                    tile_eid, ntiles)
    y0 = y_sorted[p0]
    y1 = y_sorted[p1]
    shg = _shexp(x, shared_gate_w, shared_up_w, shared_down_w,
                 shared_expert_gate_w)

    final = _combine(y0, y1, topv, shg)
    return final.reshape(Bv, Sv, Hd), logits


# speedup vs baseline: 1.3181x; 1.3181x over previous
import functools

import jax
import jax.numpy as jnp
from jax import lax
from jax.experimental import pallas as pl
from jax.experimental.pallas import tpu as pltpu
from jax.experimental.pallas import tpu_sc as plsc

N = 4096
H = 768
E = 64
K = 2
MOE_FF = 768
FF = 1536

TM = 256
NT_PAD = (N * K) // TM + E
NP = NT_PAD * TM

TR = 512
TB = 256
TC_ = 512


def _router_body(x_ref, gw_ref, logits_ref, topv_ref, topi_ref, counts_ref,
                 acc_ref):
    i = pl.program_id(0)

    @pl.when(i == 0)
    def _():
        acc_ref[...] = jnp.zeros_like(acc_ref)

    x = x_ref[...]
    gw = gw_ref[...]
    logits = lax.dot_general(x, gw, (((1,), (1,)), ((), ())),
                             preferred_element_type=jnp.float32)
    logits_ref[...] = logits
    m = jnp.max(logits, axis=1, keepdims=True)
    p = jnp.exp(logits - m)
    probs = p / jnp.sum(p, axis=1, keepdims=True)
    cols = lax.broadcasted_iota(jnp.int32, probs.shape, 1)
    v1 = jnp.max(probs, axis=1, keepdims=True)
    i1 = jnp.min(jnp.where(probs == v1, cols, E), axis=1, keepdims=True)
    probs2 = jnp.where(cols == i1, -1.0, probs)
    v2 = jnp.max(probs2, axis=1, keepdims=True)
    i2 = jnp.min(jnp.where(probs2 == v2, cols, E), axis=1, keepdims=True)
    topv_ref[...] = jnp.concatenate([v1, v2], axis=1)
    topi_ref[...] = jnp.concatenate([i1, i2], axis=1)
    oh = ((cols == i1).astype(jnp.float32) + (cols == i2).astype(jnp.float32))
    acc_ref[...] = acc_ref[...] + jnp.sum(oh, axis=0, keepdims=True)
    counts_ref[...] = jnp.broadcast_to(acc_ref[...], counts_ref.shape)


def _router(x, gate_w):
    return pl.pallas_call(
        _router_body,
        grid=(N // TR,),
        in_specs=[
            pl.BlockSpec((TR, H), lambda i: (i, 0)),
            pl.BlockSpec((E, H), lambda i: (0, 0)),
        ],
        out_specs=[
            pl.BlockSpec((TR, E), lambda i: (i, 0)),
            pl.BlockSpec((TR, K), lambda i: (i, 0)),
            pl.BlockSpec((TR, K), lambda i: (i, 0)),
            pl.BlockSpec((8, E), lambda i: (0, 0)),
        ],
        out_shape=[
            jax.ShapeDtypeStruct((N, E), jnp.float32),
            jax.ShapeDtypeStruct((N, K), jnp.float32),
            jax.ShapeDtypeStruct((N, K), jnp.int32),
            jax.ShapeDtypeStruct((8, E), jnp.float32),
        ],
        scratch_shapes=[pltpu.VMEM((1, E), jnp.float32)],
        interpret=False,
    )(x, gate_w)


def _meta_body(topi_ref, counts_ref, ppos_ref, packed_ref, carry_ref,
               pstart_ref):
    s = pl.program_id(0)

    @pl.when(s == 0)
    def _():
        counts = counts_ref[0:1, :]
        counts_i = counts.astype(jnp.int32)
        tiles_i = (counts_i + (TM - 1)) // TM
        tiles_f = tiles_i.astype(jnp.float32)
        r64 = lax.broadcasted_iota(jnp.int32, (E, E), 0)
        c64 = lax.broadcasted_iota(jnp.int32, (E, E), 1)
        incl = (r64 <= c64).astype(jnp.float32)
        cum = lax.dot_general(tiles_f, incl, (((1,), (0,)), ((), ())),
                              preferred_element_type=jnp.float32)
        pstart_ref[...] = (cum - tiles_f) * TM
        carry_ref[...] = jnp.zeros_like(carry_ref)
        jrow = lax.broadcasted_iota(
            jnp.int32, (NT_PAD + 8, E), 0).astype(jnp.float32)
        te = jnp.sum((cum <= jrow).astype(jnp.float32), axis=1, keepdims=True)
        te = jnp.minimum(te, E - 1)
        total = jnp.sum(tiles_f)
        rid = lax.broadcasted_iota(jnp.int32, (NT_PAD + 8, 1), 0)
        packed_ref[...] = jnp.where(rid == NT_PAD, total, te).astype(jnp.int32)

    blk = topi_ref[...]
    e0 = blk[:, 0:1]
    e1 = blk[:, 1:2]
    lanes = lax.broadcasted_iota(jnp.int32, (TB, E), 1)
    oh0 = (lanes == e0).astype(jnp.float32)
    oh1 = (lanes == e1).astype(jnp.float32)
    r = lax.broadcasted_iota(jnp.int32, (TB, TB), 0)
    c = lax.broadcasted_iota(jnp.int32, (TB, TB), 1)
    ls = (c < r).astype(jnp.float32)
    base = carry_ref[...] + pstart_ref[...]
    rank0 = lax.dot_general(ls, oh0, (((1,), (0,)), ((), ())),
                            preferred_element_type=jnp.float32)
    pos0 = jnp.sum((rank0 + base) * oh0, axis=1, keepdims=True)
    sum0 = jnp.sum(oh0, axis=0, keepdims=True)
    rank1 = lax.dot_general(ls, oh1, (((1,), (0,)), ((), ())),
                            preferred_element_type=jnp.float32)
    pos1 = jnp.sum((rank1 + base + sum0) * oh1, axis=1, keepdims=True)
    ppos_ref[...] = jnp.concatenate([pos0, pos1], axis=0).astype(jnp.int32)
    carry_ref[...] = (carry_ref[...] + sum0
                      + jnp.sum(oh1, axis=0, keepdims=True))


def _meta(topi, counts8):
    return pl.pallas_call(
        _meta_body,
        grid=(N // TB,),
        in_specs=[
            pl.BlockSpec((TB, K), lambda s: (s, 0)),
            pl.BlockSpec((8, E), lambda s: (0, 0)),
        ],
        out_specs=[
            pl.BlockSpec((2 * TB, 1), lambda s: (s, 0)),
            pl.BlockSpec((NT_PAD + 8, 1), lambda s: (0, 0)),
        ],
        out_shape=[
            jax.ShapeDtypeStruct((N * K, 1), jnp.int32),
            jax.ShapeDtypeStruct((NT_PAD + 8, 1), jnp.int32),
        ],
        scratch_shapes=[pltpu.VMEM((1, E), jnp.float32),
                        pltpu.VMEM((1, E), jnp.float32)],
        interpret=False,
    )(topi, counts8)


CHUNK = 64
NCHUNK = (N * K) // CHUNK
NW = 32
CPW = NCHUNK // NW


def _dispatch_body(x_hbm, ppos_hbm, out_hbm, idx0, idx1, rows0, rows1,
                   sem0, sem1):
    wid = lax.axis_index("s") * 2 + lax.axis_index("c")
    idx = (idx0, idx1)
    rows = (rows0, rows1)
    sems = (sem0, sem1)
    pending = [None, None]
    for c in range(CPW):
        b = c % 2
        if pending[b] is not None:
            pending[b].wait()
        g = wid * CPW + c
        a0 = g * CHUNK
        t0 = (a0 // (K * TB)) * TB + (a0 % (K * TB)) % TB
        t0 = pl.multiple_of(t0, CHUNK)
        pltpu.sync_copy(ppos_hbm.at[g], idx[b])
        pltpu.sync_copy(x_hbm.at[pl.ds(t0, CHUNK)], rows[b])
        pending[b] = pltpu.async_copy(rows[b], out_hbm.at[idx[b]], sems[b])
    for p in pending:
        if p is not None:
            p.wait()


def _dispatch(x, ppos_r):
    fn = pl.kernel(
        _dispatch_body,
        mesh=plsc.VectorSubcoreMesh(core_axis_name="c", subcore_axis_name="s"),
        out_type=jax.ShapeDtypeStruct((NP, H), jnp.float32),
        scratch_types=[
            pltpu.VMEM((CHUNK,), jnp.int32),
            pltpu.VMEM((CHUNK,), jnp.int32),
            pltpu.VMEM((CHUNK, H), jnp.float32),
            pltpu.VMEM((CHUNK, H), jnp.float32),
            pltpu.SemaphoreType.DMA,
            pltpu.SemaphoreType.DMA,
        ],
    )
    return fn(x, ppos_r)


def _gmm_body(tile_eid_ref, ntiles_ref, x_ref, wg_ref, wu_ref, wd_ref, y_ref):
    i = pl.program_id(0)

    @pl.when(i < ntiles_ref[0])
    def _():
        x = x_ref[...].astype(jnp.bfloat16)
        g = lax.dot_general(x, wg_ref[0].astype(jnp.bfloat16),
                            (((1,), (1,)), ((), ())),
                            preferred_element_type=jnp.float32)
        u = lax.dot_general(x, wu_ref[0].astype(jnp.bfloat16),
                            (((1,), (1,)), ((), ())),
                            preferred_element_type=jnp.float32)
        h = (g * jax.nn.sigmoid(g) * u).astype(jnp.bfloat16)
        y_ref[...] = lax.dot_general(h, wd_ref[0].astype(jnp.bfloat16),
                                     (((1,), (1,)), ((), ())),
                                     preferred_element_type=jnp.float32)


def _gmm(x_sorted, wg, wu, wd, tile_eid, ntiles):
    def _ci(i, nt):
        return jnp.minimum(i, nt[0] - 1)

    grid_spec = pltpu.PrefetchScalarGridSpec(
        num_scalar_prefetch=2,
        grid=(NT_PAD,),
        in_specs=[
            pl.BlockSpec((TM, H), lambda i, eid, nt: (_ci(i, nt), 0)),
            pl.BlockSpec((1, MOE_FF, H),
                         lambda i, eid, nt: (eid[_ci(i, nt)], 0, 0)),
            pl.BlockSpec((1, MOE_FF, H),
                         lambda i, eid, nt: (eid[_ci(i, nt)], 0, 0)),
            pl.BlockSpec((1, H, MOE_FF),
                         lambda i, eid, nt: (eid[_ci(i, nt)], 0, 0)),
        ],
        out_specs=pl.BlockSpec((TM, H), lambda i, eid, nt: (_ci(i, nt), 0)),
    )
    return pl.pallas_call(
        _gmm_body,
        grid_spec=grid_spec,
        out_shape=jax.ShapeDtypeStruct((NP, H), jnp.float32),
        interpret=False,
    )(tile_eid, ntiles, x_sorted, wg, wu, wd)


def _shexp_body(x_ref, sg_ref, su_ref, sd_ref, seg_ref, out_ref):
    x = x_ref[...]
    xb = x.astype(jnp.bfloat16)
    g = lax.dot_general(xb, sg_ref[...].astype(jnp.bfloat16),
                        (((1,), (1,)), ((), ())),
                        preferred_element_type=jnp.float32)
    u = lax.dot_general(xb, su_ref[...].astype(jnp.bfloat16),
                        (((1,), (1,)), ((), ())),
                        preferred_element_type=jnp.float32)
    h = (g * jax.nn.sigmoid(g) * u).astype(jnp.bfloat16)
    sh = lax.dot_general(h, sd_ref[...].astype(jnp.bfloat16),
                         (((1,), (1,)), ((), ())),
                         preferred_element_type=jnp.float32)
    gate = jax.nn.sigmoid(
        lax.dot_general(x, seg_ref[...], (((1,), (1,)), ((), ())),
                        preferred_element_type=jnp.float32))
    out_ref[...] = gate * sh


def _shexp(x, sg, su, sd, seg):
    return pl.pallas_call(
        _shexp_body,
        grid=(N // TC_,),
        in_specs=[
            pl.BlockSpec((TC_, H), lambda i: (i, 0)),
            pl.BlockSpec((FF, H), lambda i: (0, 0)),
            pl.BlockSpec((FF, H), lambda i: (0, 0)),
            pl.BlockSpec((H, FF), lambda i: (0, 0)),
            pl.BlockSpec((1, H), lambda i: (0, 0)),
        ],
        out_specs=pl.BlockSpec((TC_, H), lambda i: (i, 0)),
        out_shape=jax.ShapeDtypeStruct((N, H), jnp.float32),
        interpret=False,
    )(x, sg, su, sd, seg)


def _combine_body(y0_ref, y1_ref, topv_ref, shg_ref, out_ref):
    topv = topv_ref[...]
    out_ref[...] = (topv[:, 0:1] * y0_ref[...] + topv[:, 1:2] * y1_ref[...]
                    + shg_ref[...])


def _combine(y0, y1, topv, shg):
    return pl.pallas_call(
        _combine_body,
        grid=(N // TC_,),
        in_specs=[
            pl.BlockSpec((TC_, H), lambda i: (i, 0)),
            pl.BlockSpec((TC_, H), lambda i: (i, 0)),
            pl.BlockSpec((TC_, K), lambda i: (i, 0)),
            pl.BlockSpec((TC_, H), lambda i: (i, 0)),
        ],
        out_specs=pl.BlockSpec((TC_, H), lambda i: (i, 0)),
        out_shape=jax.ShapeDtypeStruct((N, H), jnp.float32),
        interpret=False,
    )(y0, y1, topv, shg)


def kernel(hidden_states, gate_w, expert_gate_w, expert_up_w, expert_down_w,
           shared_gate_w, shared_up_w, shared_down_w, shared_expert_gate_w):
    Bv, Sv, Hd = hidden_states.shape
    x = hidden_states.reshape(-1, Hd)

    logits, topv, topi, counts8 = _router(x, gate_w)
    ppos_col, packed = _meta(topi, counts8)

    ppos_r = ppos_col.reshape(N // TB, K, TB)
    p0 = ppos_r[:, 0, :].reshape(N)
    p1 = ppos_r[:, 1, :].reshape(N)
    tile_eid = packed[:NT_PAD, 0]
    ntiles = packed[NT_PAD:NT_PAD + 1, 0]

    x_sorted = _dispatch(x, ppos_col.reshape(NCHUNK, CHUNK))
    y_sorted = _gmm(x_sorted, expert_gate_w, expert_up_w, expert_down_w,
                    tile_eid, ntiles)
    y0 = y_sorted[p0]
    y1 = y_sorted[p1]
    shg = _shexp(x, shared_gate_w, shared_up_w, shared_down_w,
                 shared_expert_gate_w)

    final = _combine(y0, y1, topv, shg)
    return final.reshape(Bv, Sv, Hd), logits
